# baseline (speedup 1.0000x reference)
import functools

import jax
import jax.numpy as jnp
from jax import lax
from jax.experimental import pallas as pl
from jax.experimental.pallas import tpu as pltpu
from jax.experimental.pallas import tpu_sc as plsc

B, S = 4096, 200
D = 64
DP = 128
L = 16
NC, NS = 2, 16
NW = NC * NS
W = 128
NB = 3
NT = 2

_mesh = plsc.VectorSubcoreMesh(core_axis_name="c", subcore_axis_name="s")


@functools.partial(
    pl.kernel,
    out_type=jax.ShapeDtypeStruct((S, D, B), jnp.float32),
    mesh=_mesh,
    scratch_types=[
        pltpu.VMEM((S, W), jnp.int32),
        pltpu.VMEM((NB, W, DP), jnp.float32),
        pltpu.VMEM((NT, D, W), jnp.float32),
        pltpu.SemaphoreType.DMA((NB,)),
        pltpu.SemaphoreType.DMA((NT,)),
        pltpu.SemaphoreType.DMA,
    ],
    compiler_params=pltpu.CompilerParams(
        use_tc_tiling_on_sc=True, needs_layout_passes=False
    ),
)
def _gather_kernel(ids_hbm, table_hbm, out_hbm, idx_v, slab_v, tb_v,
                   gsem, ssem, isem):
    wid = lax.axis_index("s") * NC + lax.axis_index("c")
    col0 = wid * W

    pltpu.async_copy(
        ids_hbm.at[:, pl.ds(col0, W)], idx_v, isem
    ).wait()

    def gather(s, buf):
        return pltpu.async_copy(
            table_hbm.at[idx_v.at[s]], slab_v.at[buf], gsem.at[buf]
        )

    def wait_gather(s, buf):
        pltpu.make_async_copy(
            table_hbm.at[idx_v.at[s]], slab_v.at[buf], gsem.at[buf]
        ).wait()

    def store(s, tbuf):
        return pltpu.async_copy(
            tb_v.at[tbuf], out_hbm.at[s, :, pl.ds(col0, W)], ssem.at[tbuf]
        )

    def wait_store(s, tbuf):
        pltpu.make_async_copy(
            tb_v.at[tbuf], out_hbm.at[s, :, pl.ds(col0, W)], ssem.at[tbuf]
        ).wait()

    iotas = [lax.iota(jnp.int32, L) + (L * k) for k in range(W // L)]

    def transpose(buf, tbuf):
        slab = slab_v.at[buf]
        tb = tb_v.at[tbuf]

        def drow(d, _):
            dv = jnp.full((L,), d, jnp.int32)
            for k in range(W // L):
                v = plsc.load_gather(slab, [iotas[k], dv])
                tb[d, pl.ds(L * k, L)] = v
            return 0

        lax.fori_loop(0, D, drow, 0)

    for b in range(NB - 1):
        gather(b, b)

    def step(s, _):
        buf = lax.rem(s, NB)
        tbuf = lax.rem(s, NT)

        @pl.when(s + NB - 1 < S)
        def _():
            gather(s + NB - 1, lax.rem(s + NB - 1, NB))

        wait_gather(s, buf)

        @pl.when(s >= NT)
        def _():
            wait_store(s - NT, tbuf)

        transpose(buf, tbuf)
        store(s, tbuf)
        return 0

    lax.fori_loop(0, S, step, 0)

    wait_store(S - 2, lax.rem(S - 2, NT))
    wait_store(S - 1, lax.rem(S - 1, NT))


def kernel(zone_ids, table):
    table_p = jnp.pad(table, ((0, 0), (0, DP - D)))
    out_t = _gather_kernel(zone_ids.T.astype(jnp.int32), table_p)
    return jnp.transpose(out_t, (2, 0, 1))


# speedup vs baseline: 1.1464x; 1.1464x over previous
import functools

import jax
import jax.numpy as jnp
from jax import lax
from jax.experimental import pallas as pl
from jax.experimental.pallas import tpu as pltpu
from jax.experimental.pallas import tpu_sc as plsc

B, S = 4096, 200
D = 64
DP = 128
DPB = 132
L = 16
NC, NS = 2, 16
NW = NC * NS
W = 128
NB = 2
NT = 2

_mesh = plsc.VectorSubcoreMesh(core_axis_name="c", subcore_axis_name="s")


@functools.partial(
    pl.kernel,
    out_type=jax.ShapeDtypeStruct((S, D, B), jnp.float32),
    mesh=_mesh,
    scratch_types=[
        pltpu.VMEM((S, W), jnp.int32),
        pltpu.VMEM((NB, W, DPB), jnp.float32),
        pltpu.VMEM((NT, D, W), jnp.float32),
        pltpu.SemaphoreType.DMA((NB,)),
        pltpu.SemaphoreType.DMA((NT,)),
        pltpu.SemaphoreType.DMA,
    ],
    compiler_params=pltpu.CompilerParams(
        use_tc_tiling_on_sc=True, needs_layout_passes=False
    ),
)
def _gather_kernel(ids_hbm, table_hbm, out_hbm, idx_v, slab_v, tb_v,
                   gsem, ssem, isem):
    wid = lax.axis_index("s") * NC + lax.axis_index("c")
    col0 = wid * W

    pltpu.async_copy(
        ids_hbm.at[:, pl.ds(col0, W)], idx_v, isem
    ).wait()

    def gather(s, buf):
        return pltpu.async_copy(
            table_hbm.at[idx_v.at[s]], slab_v.at[buf, :, pl.ds(0, DP)], gsem.at[buf]
        )

    def wait_gather(s, buf):
        pltpu.make_async_copy(
            table_hbm.at[idx_v.at[s]], slab_v.at[buf, :, pl.ds(0, DP)], gsem.at[buf]
        ).wait()

    def store(s, tbuf):
        return pltpu.async_copy(
            tb_v.at[tbuf], out_hbm.at[s, :, pl.ds(col0, W)], ssem.at[tbuf]
        )

    def wait_store(s, tbuf):
        pltpu.make_async_copy(
            tb_v.at[tbuf], out_hbm.at[s, :, pl.ds(col0, W)], ssem.at[tbuf]
        ).wait()

    iotas = [lax.iota(jnp.int32, L) + (L * k) for k in range(W // L)]

    def transpose(buf, tbuf):
        slab = slab_v.at[buf]
        tb = tb_v.at[tbuf]

        def drow(d, _):
            dv = jnp.full((L,), d, jnp.int32)
            vs = [
                plsc.load_gather(slab, [iotas[k], dv]) for k in range(W // L)
            ]
            for k in range(W // L):
                tb[d, pl.ds(L * k, L)] = vs[k]
            return 0

        lax.fori_loop(0, D, drow, 0)

    for b in range(NB - 1):
        gather(b, b)

    def step(s, _):
        buf = lax.rem(s, NB)
        tbuf = lax.rem(s, NT)

        @pl.when(s + NB - 1 < S)
        def _():
            gather(s + NB - 1, lax.rem(s + NB - 1, NB))

        wait_gather(s, buf)

        @pl.when(s >= NT)
        def _():
            wait_store(s - NT, tbuf)

        transpose(buf, tbuf)
        store(s, tbuf)
        return 0

    lax.fori_loop(0, S, step, 0)

    wait_store(S - 2, lax.rem(S - 2, NT))
    wait_store(S - 1, lax.rem(S - 1, NT))


def kernel(zone_ids, table):
    table_p = jnp.pad(table, ((0, 0), (0, DP - D)))
    out_t = _gather_kernel(zone_ids.T.astype(jnp.int32), table_p)
    return jnp.transpose(out_t, (2, 0, 1))


# speedup vs baseline: 1.1468x; 1.0003x over previous
import functools

import jax
import jax.numpy as jnp
from jax import lax
from jax.experimental import pallas as pl
from jax.experimental.pallas import tpu as pltpu
from jax.experimental.pallas import tpu_sc as plsc

B, S = 4096, 200
D = 64
DP = 128
DPB = 133
L = 16
NC, NS = 2, 16
NW = NC * NS
W = 128
NB = 2
NT = 2

_mesh = plsc.VectorSubcoreMesh(core_axis_name="c", subcore_axis_name="s")


@functools.partial(
    pl.kernel,
    out_type=jax.ShapeDtypeStruct((S, D, B), jnp.float32),
    mesh=_mesh,
    scratch_types=[
        pltpu.VMEM((S, W), jnp.int32),
        pltpu.VMEM((NB, W, DPB), jnp.float32),
        pltpu.VMEM((NT, D, W), jnp.float32),
        pltpu.SemaphoreType.DMA((NB,)),
        pltpu.SemaphoreType.DMA((NT,)),
        pltpu.SemaphoreType.DMA,
    ],
    compiler_params=pltpu.CompilerParams(
        use_tc_tiling_on_sc=True, needs_layout_passes=False
    ),
)
def _gather_kernel(ids_hbm, table_hbm, out_hbm, idx_v, slab_v, tb_v,
                   gsem, ssem, isem):
    wid = lax.axis_index("s") * NC + lax.axis_index("c")
    col0 = wid * W

    pltpu.async_copy(
        ids_hbm.at[:, pl.ds(col0, W)], idx_v, isem
    ).wait()

    def gather(s, buf):
        return pltpu.async_copy(
            table_hbm.at[idx_v.at[s]], slab_v.at[buf, :, pl.ds(0, DP)], gsem.at[buf]
        )

    def wait_gather(s, buf):
        pltpu.make_async_copy(
            table_hbm.at[idx_v.at[s]], slab_v.at[buf, :, pl.ds(0, DP)], gsem.at[buf]
        ).wait()

    def store(s, tbuf):
        return pltpu.async_copy(
            tb_v.at[tbuf], out_hbm.at[s, :, pl.ds(col0, W)], ssem.at[tbuf]
        )

    def wait_store(s, tbuf):
        pltpu.make_async_copy(
            tb_v.at[tbuf], out_hbm.at[s, :, pl.ds(col0, W)], ssem.at[tbuf]
        ).wait()

    iotas = [lax.iota(jnp.int32, L) + (L * k) for k in range(W // L)]

    def transpose(buf, tbuf):
        slab = slab_v.at[buf]
        tb = tb_v.at[tbuf]

        def drow(d, _):
            dv = jnp.full((L,), d, jnp.int32)
            vs = [
                plsc.load_gather(slab, [iotas[k], dv]) for k in range(W // L)
            ]
            for k in range(W // L):
                tb[d, pl.ds(L * k, L)] = vs[k]
            return 0

        lax.fori_loop(0, D, drow, 0)

    for b in range(NB - 1):
        gather(b, b)

    def step(s, _):
        buf = lax.rem(s, NB)
        tbuf = lax.rem(s, NT)

        @pl.when(s + NB - 1 < S)
        def _():
            gather(s + NB - 1, lax.rem(s + NB - 1, NB))

        wait_gather(s, buf)

        @pl.when(s >= NT)
        def _():
            wait_store(s - NT, tbuf)

        transpose(buf, tbuf)
        store(s, tbuf)
        return 0

    lax.fori_loop(0, S, step, 0)

    wait_store(S - 2, lax.rem(S - 2, NT))
    wait_store(S - 1, lax.rem(S - 1, NT))


def kernel(zone_ids, table):
    table_p = jnp.pad(table, ((0, 0), (0, DP - D)))
    out_t = _gather_kernel(zone_ids.T.astype(jnp.int32), table_p)
    return jnp.transpose(out_t, (2, 0, 1))


# speedup vs baseline: 1.7320x; 1.5104x over previous
import functools

import jax
import jax.numpy as jnp
from jax import lax
from jax.experimental import pallas as pl
from jax.experimental.pallas import tpu as pltpu
from jax.experimental.pallas import tpu_sc as plsc

B, S = 4096, 200
D = 64
DP = 128
NC, NS = 2, 16
NW = NC * NS
B_PER_W = B // NW
NB = 3
S0 = 128
S1 = S - S0

_mesh = plsc.VectorSubcoreMesh(core_axis_name="c", subcore_axis_name="s")


@functools.partial(
    pl.kernel,
    out_type=jax.ShapeDtypeStruct((B, S, DP), jnp.float32),
    mesh=_mesh,
    scratch_types=[
        pltpu.VMEM((B_PER_W, S), jnp.int32),
        pltpu.VMEM((NB, S, DP), jnp.float32),
        pltpu.SemaphoreType.DMA((NB,)),
        pltpu.SemaphoreType.DMA((NB,)),
    ],
    compiler_params=pltpu.CompilerParams(use_tc_tiling_on_sc=True),
)
def _gather_kernel(ids_hbm, table_hbm, out_hbm, idx_v, rows_v, gsem, ssem):
    wid = lax.axis_index("s") * NC + lax.axis_index("c")
    b_base = wid * B_PER_W

    pltpu.sync_copy(ids_hbm.at[pl.ds(b_base, B_PER_W)], idx_v)

    def gathers(i, buf):
        pltpu.async_copy(
            table_hbm.at[idx_v.at[i, pl.ds(0, S0)]],
            rows_v.at[buf, pl.ds(0, S0)],
            gsem.at[buf],
        )
        pltpu.async_copy(
            table_hbm.at[idx_v.at[i, pl.ds(S0, S1)]],
            rows_v.at[buf, pl.ds(S0, S1)],
            gsem.at[buf],
        )

    def wait_gathers(i, buf):
        pltpu.make_async_copy(
            table_hbm.at[idx_v.at[i, pl.ds(0, S0)]],
            rows_v.at[buf, pl.ds(0, S0)],
            gsem.at[buf],
        ).wait()
        pltpu.make_async_copy(
            table_hbm.at[idx_v.at[i, pl.ds(S0, S1)]],
            rows_v.at[buf, pl.ds(S0, S1)],
            gsem.at[buf],
        ).wait()

    def store(i, buf):
        return pltpu.async_copy(
            rows_v.at[buf], out_hbm.at[b_base + i], ssem.at[buf]
        )

    def wait_store(i, buf):
        pltpu.make_async_copy(
            rows_v.at[buf], out_hbm.at[b_base + i], ssem.at[buf]
        ).wait()

    for b in range(NB):
        gathers(b, b)

    n_groups = B_PER_W // NB

    def group(g, _):
        base = g * NB
        for b in range(NB):
            i = base + b
            wait_gathers(i, b)
            store(i, b)
        for b in range(NB):
            i = base + b
            wait_store(i, b)

            @pl.when(i + NB < B_PER_W)
            def _():
                gathers(i + NB, b)

        return 0

    lax.fori_loop(0, n_groups, group, 0)

    for i in range(n_groups * NB, B_PER_W):
        b = i % NB
        wait_gathers(i, b)
        store(i, b).wait()


def kernel(zone_ids, table):
    table_p = jnp.pad(table, ((0, 0), (0, DP - D)))
    out_p = _gather_kernel(zone_ids.astype(jnp.int32), table_p)
    return out_p[:, :, :D]


# speedup vs baseline: 1.8559x; 1.0715x over previous
import functools

import jax
import jax.numpy as jnp
from jax import lax
from jax.experimental import pallas as pl
from jax.experimental.pallas import tpu as pltpu
from jax.experimental.pallas import tpu_sc as plsc

B, S = 4096, 200
D = 64
DP = 128
NC, NS = 2, 16
NW = NC * NS
B_PER_W = B // NW
NB = 3
S0 = 128
S1 = S - S0

_mesh = plsc.VectorSubcoreMesh(core_axis_name="c", subcore_axis_name="s")


@functools.partial(
    pl.kernel,
    out_type=jax.ShapeDtypeStruct((B, S, DP), jnp.float32),
    mesh=_mesh,
    scratch_types=[
        pltpu.VMEM((B_PER_W, S), jnp.int32),
        pltpu.VMEM((NB, S, DP), jnp.float32),
        pltpu.SemaphoreType.DMA((NB,)),
        pltpu.SemaphoreType.DMA((NB,)),
    ],
    compiler_params=pltpu.CompilerParams(use_tc_tiling_on_sc=True),
)
def _gather_kernel(ids_hbm, table_hbm, out_hbm, idx_v, rows_v, gsem, ssem):
    wid = lax.axis_index("s") * NC + lax.axis_index("c")
    b_base = wid * B_PER_W

    pltpu.sync_copy(ids_hbm.at[pl.ds(b_base, B_PER_W)], idx_v)

    def gathers(i, buf):
        pltpu.async_copy(
            table_hbm.at[idx_v.at[i, pl.ds(0, S0)]],
            rows_v.at[buf, pl.ds(0, S0)],
            gsem.at[buf],
        )
        pltpu.async_copy(
            table_hbm.at[idx_v.at[i, pl.ds(S0, S1)]],
            rows_v.at[buf, pl.ds(S0, S1)],
            gsem.at[buf],
        )

    def wait_gathers(i, buf):
        pltpu.make_async_copy(
            table_hbm.at[idx_v.at[i, pl.ds(0, S0)]],
            rows_v.at[buf, pl.ds(0, S0)],
            gsem.at[buf],
        ).wait()
        pltpu.make_async_copy(
            table_hbm.at[idx_v.at[i, pl.ds(S0, S1)]],
            rows_v.at[buf, pl.ds(S0, S1)],
            gsem.at[buf],
        ).wait()

    def store(i, buf):
        return pltpu.async_copy(
            rows_v.at[buf], out_hbm.at[b_base + i], ssem.at[buf]
        )

    def wait_store(i, buf):
        pltpu.make_async_copy(
            rows_v.at[buf], out_hbm.at[b_base + i], ssem.at[buf]
        ).wait()

    for b in range(NB):
        gathers(b, b)

    n_groups = B_PER_W // NB

    def group(g, _):
        base = g * NB
        for b in range(NB):
            i = base + b
            wait_gathers(i, b)
            store(i, b)
        for b in range(NB):
            i = base + b
            wait_store(i, b)

            @pl.when(i + NB < B_PER_W)
            def _():
                gathers(i + NB, b)

        return 0

    lax.fori_loop(0, n_groups, group, 0)

    for i in range(n_groups * NB, B_PER_W):
        b = i % NB
        wait_gathers(i, b)
        store(i, b).wait()


N_ROWS = 1000000
CB = 2048


def _pad_transpose_body(in_ref, out_ref):
    x = in_ref[...]
    out_ref[:, :D] = x.T
    out_ref[:, D:] = jnp.zeros((CB, DP - D), jnp.float32)


def _pad_transpose(table_t):
    return pl.pallas_call(
        _pad_transpose_body,
        grid=(pl.cdiv(N_ROWS, CB),),
        in_specs=[pl.BlockSpec((D, CB), lambda j: (0, j))],
        out_specs=pl.BlockSpec((CB, DP), lambda j: (j, 0)),
        out_shape=jax.ShapeDtypeStruct((N_ROWS, DP), jnp.float32),
    )(table_t)


def kernel(zone_ids, table):
    table_p = _pad_transpose(table.T)
    out_p = _gather_kernel(zone_ids.astype(jnp.int32), table_p)
    return out_p[:, :, :D]


# speedup vs baseline: 2.3344x; 1.2578x over previous
import functools

import jax
import jax.numpy as jnp
from jax import lax
from jax.experimental import pallas as pl
from jax.experimental.pallas import tpu as pltpu
from jax.experimental.pallas import tpu_sc as plsc

B, S = 4096, 200
D = 64
DP = 128
NC, NS = 2, 16
NW = NC * NS
B_PER_W = B // NW
NB = 3
S0 = 128
S1 = S - S0

_mesh = plsc.VectorSubcoreMesh(core_axis_name="c", subcore_axis_name="s")


@functools.partial(
    pl.kernel,
    out_type=jax.ShapeDtypeStruct((B, S, DP), jnp.float32),
    mesh=_mesh,
    scratch_types=[
        pltpu.VMEM((B_PER_W, S), jnp.int32),
        pltpu.VMEM((NB, S, DP), jnp.float32),
        pltpu.SemaphoreType.DMA((NB,)),
        pltpu.SemaphoreType.DMA((NB,)),
    ],
    compiler_params=pltpu.CompilerParams(use_tc_tiling_on_sc=True),
)
def _gather_kernel(ids_hbm, table_hbm, out_hbm, idx_v, rows_v, gsem, ssem):
    wid = lax.axis_index("s") * NC + lax.axis_index("c")
    b_base = wid * B_PER_W

    pltpu.sync_copy(ids_hbm.at[pl.ds(b_base, B_PER_W)], idx_v)

    def gathers(i, buf):
        pltpu.async_copy(
            table_hbm.at[idx_v.at[i, pl.ds(0, S0)]],
            rows_v.at[buf, pl.ds(0, S0)],
            gsem.at[buf],
        )
        pltpu.async_copy(
            table_hbm.at[idx_v.at[i, pl.ds(S0, S1)]],
            rows_v.at[buf, pl.ds(S0, S1)],
            gsem.at[buf],
        )

    def wait_gathers(i, buf):
        pltpu.make_async_copy(
            table_hbm.at[idx_v.at[i, pl.ds(0, S0)]],
            rows_v.at[buf, pl.ds(0, S0)],
            gsem.at[buf],
        ).wait()
        pltpu.make_async_copy(
            table_hbm.at[idx_v.at[i, pl.ds(S0, S1)]],
            rows_v.at[buf, pl.ds(S0, S1)],
            gsem.at[buf],
        ).wait()

    def store(i, buf):
        return pltpu.async_copy(
            rows_v.at[buf], out_hbm.at[b_base + i], ssem.at[buf]
        )

    def wait_store(i, buf):
        pltpu.make_async_copy(
            rows_v.at[buf], out_hbm.at[b_base + i], ssem.at[buf]
        ).wait()

    for b in range(NB):
        gathers(b, b)

    n_groups = B_PER_W // NB

    def group(g, _):
        base = g * NB
        for b in range(NB):
            i = base + b
            wait_gathers(i, b)
            store(i, b)
        for b in range(NB):
            i = base + b
            wait_store(i, b)

            @pl.when(i + NB < B_PER_W)
            def _():
                gathers(i + NB, b)

        return 0

    lax.fori_loop(0, n_groups, group, 0)

    for i in range(n_groups * NB, B_PER_W):
        b = i % NB
        wait_gathers(i, b)
        store(i, b).wait()


N_ROWS = 1000000
CB = 8192


def _pad_transpose_body(in_ref, out_ref):
    x = in_ref[...]
    out_ref[:, :D] = x.T
    out_ref[:, D:] = jnp.zeros((CB, DP - D), jnp.float32)


def _pad_transpose(table_t):
    return pl.pallas_call(
        _pad_transpose_body,
        grid=(pl.cdiv(N_ROWS, CB),),
        in_specs=[pl.BlockSpec((D, CB), lambda j: (0, j))],
        out_specs=pl.BlockSpec((CB, DP), lambda j: (j, 0)),
        out_shape=jax.ShapeDtypeStruct((N_ROWS, DP), jnp.float32),
    )(table_t)


def kernel(zone_ids, table):
    table_p = _pad_transpose(table.T)
    out_p = _gather_kernel(zone_ids.astype(jnp.int32), table_p)
    return out_p[:, :, :D]


# speedup vs baseline: 2.4044x; 1.0300x over previous
import functools

import jax
import jax.numpy as jnp
from jax import lax
from jax.experimental import pallas as pl
from jax.experimental.pallas import tpu as pltpu
from jax.experimental.pallas import tpu_sc as plsc

B, S = 4096, 200
D = 64
DP = 128
NC, NS = 2, 16
NW = NC * NS
B_PER_W = B // NW
NB = 3
S0 = 128
S1 = S - S0

_mesh = plsc.VectorSubcoreMesh(core_axis_name="c", subcore_axis_name="s")


@functools.partial(
    pl.kernel,
    out_type=jax.ShapeDtypeStruct((B, S, DP), jnp.float32),
    mesh=_mesh,
    scratch_types=[
        pltpu.VMEM((B_PER_W, S), jnp.int32),
        pltpu.VMEM((NB, S, DP), jnp.float32),
        pltpu.SemaphoreType.DMA((NB,)),
        pltpu.SemaphoreType.DMA((NB,)),
    ],
    compiler_params=pltpu.CompilerParams(use_tc_tiling_on_sc=True),
)
def _gather_kernel(ids_hbm, table_hbm, out_hbm, idx_v, rows_v, gsem, ssem):
    wid = lax.axis_index("s") * NC + lax.axis_index("c")
    b_base = wid * B_PER_W

    pltpu.sync_copy(ids_hbm.at[pl.ds(b_base, B_PER_W)], idx_v)

    def gathers(i, buf):
        pltpu.async_copy(
            table_hbm.at[idx_v.at[i, pl.ds(0, S0)]],
            rows_v.at[buf, pl.ds(0, S0)],
            gsem.at[buf],
        )
        pltpu.async_copy(
            table_hbm.at[idx_v.at[i, pl.ds(S0, S1)]],
            rows_v.at[buf, pl.ds(S0, S1)],
            gsem.at[buf],
        )

    def wait_gathers(i, buf):
        pltpu.make_async_copy(
            table_hbm.at[idx_v.at[i, pl.ds(0, S0)]],
            rows_v.at[buf, pl.ds(0, S0)],
            gsem.at[buf],
        ).wait()
        pltpu.make_async_copy(
            table_hbm.at[idx_v.at[i, pl.ds(S0, S1)]],
            rows_v.at[buf, pl.ds(S0, S1)],
            gsem.at[buf],
        ).wait()

    def store(i, buf):
        return pltpu.async_copy(
            rows_v.at[buf], out_hbm.at[b_base + i], ssem.at[buf]
        )

    def wait_store(i, buf):
        pltpu.make_async_copy(
            rows_v.at[buf], out_hbm.at[b_base + i], ssem.at[buf]
        ).wait()

    for b in range(NB):
        gathers(b, b)

    n_groups = B_PER_W // NB

    def group(g, _):
        base = g * NB
        for b in range(NB):
            i = base + b
            wait_gathers(i, b)
            store(i, b)
        for b in range(NB):
            i = base + b
            wait_store(i, b)

            @pl.when(i + NB < B_PER_W)
            def _():
                gathers(i + NB, b)

        return 0

    lax.fori_loop(0, n_groups, group, 0)

    for i in range(n_groups * NB, B_PER_W):
        b = i % NB
        wait_gathers(i, b)
        store(i, b).wait()


N_ROWS = 1000000
CB = 16384


def _pad_transpose_body(in_ref, out_ref):
    x = in_ref[...]
    out_ref[:, :D] = x.T
    out_ref[:, D:] = jnp.zeros((CB, DP - D), jnp.float32)


def _pad_transpose(table_t):
    return pl.pallas_call(
        _pad_transpose_body,
        grid=(pl.cdiv(N_ROWS, CB),),
        in_specs=[pl.BlockSpec((D, CB), lambda j: (0, j))],
        out_specs=pl.BlockSpec((CB, DP), lambda j: (j, 0)),
        out_shape=jax.ShapeDtypeStruct((N_ROWS, DP), jnp.float32),
    )(table_t)


def kernel(zone_ids, table):
    table_p = _pad_transpose(table.T)
    out_p = _gather_kernel(zone_ids.astype(jnp.int32), table_p)
    return out_p[:, :, :D]


# speedup vs baseline: 2.4266x; 1.0092x over previous
import functools

import jax
import jax.numpy as jnp
from jax import lax
from jax.experimental import pallas as pl
from jax.experimental.pallas import tpu as pltpu
from jax.experimental.pallas import tpu_sc as plsc

B, S = 4096, 200
D = 64
DP = 128
NC, NS = 2, 16
NW = NC * NS
B_PER_W = B // NW
NB = 3
S0 = 128
S1 = S - S0

_mesh = plsc.VectorSubcoreMesh(core_axis_name="c", subcore_axis_name="s")


@functools.partial(
    pl.kernel,
    out_type=jax.ShapeDtypeStruct((B, S, DP), jnp.float32),
    mesh=_mesh,
    scratch_types=[
        pltpu.VMEM((B_PER_W, S), jnp.int32),
        pltpu.VMEM((NB, S, DP), jnp.float32),
        pltpu.SemaphoreType.DMA((NB,)),
        pltpu.SemaphoreType.DMA((NB,)),
    ],
    compiler_params=pltpu.CompilerParams(use_tc_tiling_on_sc=True),
)
def _gather_kernel(ids_hbm, table_hbm, out_hbm, idx_v, rows_v, gsem, ssem):
    wid = lax.axis_index("s") * NC + lax.axis_index("c")
    b_base = wid * B_PER_W

    pltpu.sync_copy(ids_hbm.at[pl.ds(b_base, B_PER_W)], idx_v)

    def gathers(i, buf):
        pltpu.async_copy(
            table_hbm.at[idx_v.at[i, pl.ds(0, S0)]],
            rows_v.at[buf, pl.ds(0, S0)],
            gsem.at[buf],
        )
        pltpu.async_copy(
            table_hbm.at[idx_v.at[i, pl.ds(S0, S1)]],
            rows_v.at[buf, pl.ds(S0, S1)],
            gsem.at[buf],
        )

    def wait_gathers(i, buf):
        pltpu.make_async_copy(
            table_hbm.at[idx_v.at[i, pl.ds(0, S0)]],
            rows_v.at[buf, pl.ds(0, S0)],
            gsem.at[buf],
        ).wait()
        pltpu.make_async_copy(
            table_hbm.at[idx_v.at[i, pl.ds(S0, S1)]],
            rows_v.at[buf, pl.ds(S0, S1)],
            gsem.at[buf],
        ).wait()

    def store(i, buf):
        return pltpu.async_copy(
            rows_v.at[buf], out_hbm.at[b_base + i], ssem.at[buf]
        )

    def wait_store(i, buf):
        pltpu.make_async_copy(
            rows_v.at[buf], out_hbm.at[b_base + i], ssem.at[buf]
        ).wait()

    for b in range(NB):
        gathers(b, b)

    n_groups = B_PER_W // NB

    def group(g, _):
        base = g * NB
        for b in range(NB):
            i = base + b
            wait_gathers(i, b)
            store(i, b)
        for b in range(NB):
            i = base + b
            wait_store(i, b)

            @pl.when(i + NB < B_PER_W)
            def _():
                gathers(i + NB, b)

        return 0

    lax.fori_loop(0, n_groups, group, 0)

    for i in range(n_groups * NB, B_PER_W):
        b = i % NB
        wait_gathers(i, b)
        store(i, b).wait()


N_ROWS = 1000000
CB = 32768


def _pad_transpose_body(in_ref, out_ref):
    x = in_ref[...]
    out_ref[:, :D] = x.T
    out_ref[:, D:] = jnp.zeros((CB, DP - D), jnp.float32)


def _pad_transpose(table_t):
    return pl.pallas_call(
        _pad_transpose_body,
        grid=(pl.cdiv(N_ROWS, CB),),
        in_specs=[pl.BlockSpec((D, CB), lambda j: (0, j))],
        out_specs=pl.BlockSpec((CB, DP), lambda j: (j, 0)),
        out_shape=jax.ShapeDtypeStruct((N_ROWS, DP), jnp.float32),
    )(table_t)


def kernel(zone_ids, table):
    table_p = _pad_transpose(table.T)
    out_p = _gather_kernel(zone_ids.astype(jnp.int32), table_p)
    return out_p[:, :, :D]
